# Initial kernel scaffold; baseline (speedup 1.0000x reference)
import jax
import jax.numpy as jnp
from jax.experimental import pallas as pl


def kernel(tokens, emb, wih1, whh1, b1, wih2, whh2, b2, wd, bd):
    raise NotImplementedError("write your pallas kernel here")


# speedup vs baseline: 9.3601x; 9.3601x over previous
import jax
import jax.numpy as jnp
from jax import lax
from jax.experimental import pallas as pl
from jax.experimental.pallas import tpu as pltpu


def _round_up(x, m):
    return ((x + m - 1) // m) * m


def _lstm_body(tok_ref,
               table_ref,
               whh1_ref,
               wih2_ref, whh2_ref, b2_ref,
               wd_ref, bd_ref,
               logits_ref, hn_ref, cn_ref,
               seq1_ref):
    BT, T = tok_ref.shape
    H = whh1_ref.shape[0]
    Lp = table_ref.shape[0]
    V = logits_ref.shape[2]
    bf16 = jnp.bfloat16
    f32 = jnp.float32

    def sig(x):
        return 0.5 * jnp.tanh(0.5 * x) + 0.5

    def act(gates, c):
        i = sig(gates[:, 0 * H:1 * H])
        f = sig(gates[:, 1 * H:2 * H])
        g = jnp.tanh(gates[:, 2 * H:3 * H])
        o = sig(gates[:, 3 * H:4 * H])
        c_new = f * c + i * g
        h_new = o * jnp.tanh(c_new)
        return h_new, c_new

    lane_iota = lax.broadcasted_iota(jnp.int32, (BT, Lp), 1)

    h = jnp.zeros((BT, H), f32)
    c = jnp.zeros((BT, H), f32)
    for t in range(T):
        oh = (lane_iota == tok_ref[:, t:t + 1]).astype(bf16)
        gx = jnp.dot(oh, table_ref[...], preferred_element_type=f32)
        gates = gx + jnp.dot(h.astype(bf16), whh1_ref[...],
                             preferred_element_type=f32)
        h, c = act(gates, c)
        seq1_ref[t * BT:(t + 1) * BT, :] = h.astype(bf16)

    for t in range(T):
        h1t = seq1_ref[t * BT:(t + 1) * BT, :]
        gx2 = jnp.dot(h1t, wih2_ref[...], preferred_element_type=f32)
        gates = gx2 + jnp.dot(h.astype(bf16), whh2_ref[...],
                              preferred_element_type=f32) + b2_ref[...]
        h, c = act(gates, c)
        lg = jnp.dot(h.astype(bf16), wd_ref[...],
                     preferred_element_type=f32) + bd_ref[...]
        logits_ref[:, t, :] = lg[:, :V]

    hn_ref[...] = h
    cn_ref[...] = c


def kernel(tokens, emb, wih1, whh1, b1, wih2, whh2, b2, wd, bd):
    B, T = tokens.shape
    V, E = emb.shape
    H = whh1.shape[0]

    BT = 256
    Bp = _round_up(B, BT)
    NB = Bp // BT
    Vp = _round_up(V, 128)
    Lp = _round_up(V, 128)

    table = jnp.pad(emb @ wih1 + b1, ((0, Lp - V), (0, 0))).astype(jnp.bfloat16)
    whh1b = whh1.astype(jnp.bfloat16)
    wih2b = wih2.astype(jnp.bfloat16)
    whh2b = whh2.astype(jnp.bfloat16)
    wdp = jnp.pad(wd, ((0, 0), (0, Vp - V))).astype(jnp.bfloat16)
    bdp = jnp.pad(bd, ((0, 0), (0, Vp - V)))
    toks = jnp.pad(tokens, ((0, Bp - B), (0, 0)))

    def full(shape):
        return pl.BlockSpec(shape, lambda b: (0,) * len(shape))

    logits, h_n, c_n = pl.pallas_call(
        _lstm_body,
        grid=(NB,),
        in_specs=[
            pl.BlockSpec((BT, T), lambda b: (b, 0)),
            full((Lp, 4 * H)), full((H, 4 * H)),
            full((H, 4 * H)), full((H, 4 * H)), full((1, 4 * H)),
            full((H, Vp)), full((1, Vp)),
        ],
        out_specs=(
            pl.BlockSpec((BT, T, V), lambda b: (b, 0, 0)),
            pl.BlockSpec((BT, H), lambda b: (b, 0)),
            pl.BlockSpec((BT, H), lambda b: (b, 0)),
        ),
        out_shape=(
            jax.ShapeDtypeStruct((Bp, T, V), jnp.float32),
            jax.ShapeDtypeStruct((Bp, H), jnp.float32),
            jax.ShapeDtypeStruct((Bp, H), jnp.float32),
        ),
        scratch_shapes=[pltpu.VMEM((T * BT, H), jnp.bfloat16)],
        compiler_params=pltpu.CompilerParams(dimension_semantics=("parallel",)),
    )(toks, table, whh1b, wih2b, whh2b, b2, wdp, bdp)

    logits = logits[:B]
    h_n = h_n[None, :B, :]
    c_n = c_n[None, :B, :]
    return logits, (h_n, c_n)


# speedup vs baseline: 9.7008x; 1.0364x over previous
import jax
import jax.numpy as jnp
from jax import lax
from jax.experimental import pallas as pl
from jax.experimental.pallas import tpu as pltpu


def _round_up(x, m):
    return ((x + m - 1) // m) * m


def _lstm_body(tok_ref,
               table_ref,
               whh1_ref,
               wih2_ref, whh2_ref, b2_ref,
               wd_ref, bd_ref,
               logits_ref, hn_ref, cn_ref,
               seq1_ref):
    BT, T = tok_ref.shape
    H = whh1_ref.shape[0]
    Lp = table_ref.shape[0]
    V = logits_ref.shape[2]
    bf16 = jnp.bfloat16
    f32 = jnp.float32

    def sig(x):
        return 0.5 * jnp.tanh(0.5 * x) + 0.5

    def act(gates, c):
        i = sig(gates[:, 0 * H:1 * H])
        f = sig(gates[:, 1 * H:2 * H])
        g = jnp.tanh(gates[:, 2 * H:3 * H])
        o = sig(gates[:, 3 * H:4 * H])
        c_new = f * c + i * g
        h_new = o * jnp.tanh(c_new)
        return h_new, c_new

    S = BT // 2
    lane_iota = lax.broadcasted_iota(jnp.int32, (S, Lp), 1)

    hA = jnp.zeros((S, H), f32)
    cA = jnp.zeros((S, H), f32)
    hB = jnp.zeros((S, H), f32)
    cB = jnp.zeros((S, H), f32)
    for t in range(T):
        ohA = (lane_iota == tok_ref[0:S, t:t + 1]).astype(bf16)
        ohB = (lane_iota == tok_ref[S:BT, t:t + 1]).astype(bf16)
        gxA = jnp.dot(ohA, table_ref[...], preferred_element_type=f32)
        gxB = jnp.dot(ohB, table_ref[...], preferred_element_type=f32)
        dA = jnp.dot(hA.astype(bf16), whh1_ref[...], preferred_element_type=f32)
        dB = jnp.dot(hB.astype(bf16), whh1_ref[...], preferred_element_type=f32)
        hA, cA = act(gxA + dA, cA)
        hB, cB = act(gxB + dB, cB)
        r0 = t * BT
        seq1_ref[r0:r0 + S, :] = hA.astype(bf16)
        seq1_ref[r0 + S:r0 + BT, :] = hB.astype(bf16)

    for t in range(T):
        r0 = t * BT
        h1A = seq1_ref[r0:r0 + S, :]
        h1B = seq1_ref[r0 + S:r0 + BT, :]
        gxA = jnp.dot(h1A, wih2_ref[...], preferred_element_type=f32) + b2_ref[...]
        gxB = jnp.dot(h1B, wih2_ref[...], preferred_element_type=f32) + b2_ref[...]
        dA = jnp.dot(hA.astype(bf16), whh2_ref[...], preferred_element_type=f32)
        dB = jnp.dot(hB.astype(bf16), whh2_ref[...], preferred_element_type=f32)
        hA, cA = act(gxA + dA, cA)
        hB, cB = act(gxB + dB, cB)
        lgA = jnp.dot(hA.astype(bf16), wd_ref[...],
                      preferred_element_type=f32) + bd_ref[...]
        lgB = jnp.dot(hB.astype(bf16), wd_ref[...],
                      preferred_element_type=f32) + bd_ref[...]
        logits_ref[0:S, t, :] = lgA[:, :V]
        logits_ref[S:BT, t, :] = lgB[:, :V]

    hn_ref[0:S, :] = hA
    hn_ref[S:BT, :] = hB
    cn_ref[0:S, :] = cA
    cn_ref[S:BT, :] = cB


def kernel(tokens, emb, wih1, whh1, b1, wih2, whh2, b2, wd, bd):
    B, T = tokens.shape
    V, E = emb.shape
    H = whh1.shape[0]

    BT = 256
    Bp = _round_up(B, BT)
    NB = Bp // BT
    Vp = _round_up(V, 128)
    Lp = _round_up(V, 128)

    table = jnp.pad(emb @ wih1 + b1, ((0, Lp - V), (0, 0))).astype(jnp.bfloat16)
    whh1b = whh1.astype(jnp.bfloat16)
    wih2b = wih2.astype(jnp.bfloat16)
    whh2b = whh2.astype(jnp.bfloat16)
    wdp = jnp.pad(wd, ((0, 0), (0, Vp - V))).astype(jnp.bfloat16)
    bdp = jnp.pad(bd, ((0, 0), (0, Vp - V)))
    toks = jnp.pad(tokens, ((0, Bp - B), (0, 0)))

    def full(shape):
        return pl.BlockSpec(shape, lambda b: (0,) * len(shape))

    logits, h_n, c_n = pl.pallas_call(
        _lstm_body,
        grid=(NB,),
        in_specs=[
            pl.BlockSpec((BT, T), lambda b: (b, 0)),
            full((Lp, 4 * H)), full((H, 4 * H)),
            full((H, 4 * H)), full((H, 4 * H)), full((1, 4 * H)),
            full((H, Vp)), full((1, Vp)),
        ],
        out_specs=(
            pl.BlockSpec((BT, T, V), lambda b: (b, 0, 0)),
            pl.BlockSpec((BT, H), lambda b: (b, 0)),
            pl.BlockSpec((BT, H), lambda b: (b, 0)),
        ),
        out_shape=(
            jax.ShapeDtypeStruct((Bp, T, V), jnp.float32),
            jax.ShapeDtypeStruct((Bp, H), jnp.float32),
            jax.ShapeDtypeStruct((Bp, H), jnp.float32),
        ),
        scratch_shapes=[pltpu.VMEM((T * BT, H), jnp.bfloat16)],
        compiler_params=pltpu.CompilerParams(dimension_semantics=("parallel",)),
    )(toks, table, whh1b, wih2b, whh2b, b2, wdp, bdp)

    logits = logits[:B]
    h_n = h_n[None, :B, :]
    c_n = c_n[None, :B, :]
    return logits, (h_n, c_n)
